# baseline (speedup 1.0000x reference)
import functools

import jax
import jax.numpy as jnp
from jax.experimental import pallas as pl
from jax.experimental.pallas import tpu as pltpu

N = 2048
A_OUT = 8
N_ATOMS = 51
V_MIN = -10.0
V_MAX = 10.0


N_CHUNKS = 8
CHUNK = N // N_CHUNKS


def _fused_kernel(feat_ref, adj_ref, mask_ref,
                  w_e1_ref, b_e1_ref, w_e2_ref, b_e2_ref,
                  w_g_ref, b_g_ref, w_gd_ref, b_gd_ref,
                  w_p1_ref, b_p1_ref, w_p2_ref, b_p2_ref,
                  w_po_ref, b_po_ref, out_ref, adj_vmem, sems):
    f32 = jnp.float32

    copies = []
    for k in range(N_CHUNKS):
        cp = pltpu.make_async_copy(
            adj_ref.at[pl.ds(k * CHUNK, CHUNK), :],
            adj_vmem.at[pl.ds(k * CHUNK, CHUNK), :],
            sems.at[k])
        cp.start()
        copies.append(cp)

    x = jnp.maximum(
        jnp.dot(feat_ref[...], w_e1_ref[...], preferred_element_type=f32)
        + b_e1_ref[...], 0.0)
    x = jnp.maximum(
        jnp.dot(x, w_e2_ref[...], preferred_element_type=f32)
        + b_e2_ref[...], 0.0)

    xw = jnp.dot(x, w_g_ref[...], preferred_element_type=f32)

    deg = jnp.zeros((1, N), f32) + 1.0
    for k in range(N_CHUNKS):
        copies[k].wait()
        deg = deg + jnp.sum(adj_vmem[pl.ds(k * CHUNK, CHUNK), :], axis=0,
                            keepdims=True)
    dinv = jax.lax.rsqrt(deg)
    z = xw * dinv.reshape(N, 1)

    agg = jax.lax.dot_general(
        adj_vmem[...], z, dimension_numbers=(((0,), (0,)), ((), ())),
        preferred_element_type=f32)
    gcn = dinv.reshape(N, 1) * (agg + z) + b_g_ref[...]
    xg = jnp.maximum(gcn, 0.0)
    xg = jnp.maximum(
        jnp.dot(xg, w_gd_ref[...], preferred_element_type=f32)
        + b_gd_ref[...], 0.0)

    p = jnp.maximum(
        jnp.dot(xg, w_p1_ref[0:32, :], preferred_element_type=f32)
        + jnp.dot(x, w_p1_ref[32:64, :], preferred_element_type=f32)
        + b_p1_ref[...], 0.0)
    p = jnp.maximum(
        jnp.dot(p, w_p2_ref[...], preferred_element_type=f32)
        + b_p2_ref[...], 0.0)
    p = (jnp.dot(p, w_po_ref[...], preferred_element_type=f32)
         + b_po_ref[...])
    p = p * mask_ref[...]

    K = A_OUT * N_ATOMS
    step = (V_MAX - V_MIN) / (N_ATOMS - 1)
    k_i = jax.lax.broadcasted_iota(jnp.int32, (K, A_OUT), 0)
    a_i = jax.lax.broadcasted_iota(jnp.int32, (K, A_OUT), 1)
    G = (k_i // N_ATOMS == a_i).astype(f32)
    a_t = jax.lax.broadcasted_iota(jnp.int32, (A_OUT, K), 0)
    k_t = jax.lax.broadcasted_iota(jnp.int32, (A_OUT, K), 1)
    Gt = (k_t // N_ATOMS == a_t).astype(f32)
    sup_row = V_MIN + (jax.lax.broadcasted_iota(
        jnp.int32, (1, K), 1) % N_ATOMS).astype(f32) * step

    def dot2(a, b):
        a_hi = a.astype(jnp.bfloat16).astype(f32)
        return (jnp.dot(a_hi, b, preferred_element_type=f32)
                + jnp.dot(a - a_hi, b, preferred_element_type=f32))

    m = jnp.max(p, axis=1, keepdims=True)
    e = jnp.exp(p - m)
    denom = dot2(e, G)
    rden = 1.0 / denom
    d = e * dot2(rden, Gt)
    d = jnp.maximum(d, 0.001)
    out_ref[...] = dot2(d * sup_row, G)


@jax.jit
def kernel(features, adjacency, mask, W_e1, b_e1, W_e2, b_e2, W_g, b_g,
           W_gd, b_gd, W_p1, b_p1, W_p2, b_p2, W_po, b_po):
    mask2 = mask.reshape(N, 1)
    row = lambda b: b.reshape(1, -1)
    n_in = 17
    in_specs = [pl.BlockSpec(memory_space=pltpu.MemorySpace.HBM) if i == 1
                else pl.BlockSpec(memory_space=pltpu.MemorySpace.VMEM)
                for i in range(n_in)]
    out = pl.pallas_call(
        _fused_kernel,
        out_shape=jax.ShapeDtypeStruct((N, A_OUT), jnp.float32),
        in_specs=in_specs,
        out_specs=pl.BlockSpec(memory_space=pltpu.MemorySpace.VMEM),
        scratch_shapes=[
            pltpu.VMEM((N, N), jnp.float32),
            pltpu.SemaphoreType.DMA((N_CHUNKS,)),
        ],
        compiler_params=pltpu.CompilerParams(
            vmem_limit_bytes=100 * 1024 * 1024),
    )(features, adjacency, mask2,
      W_e1, row(b_e1), W_e2, row(b_e2),
      W_g, row(b_g), W_gd, row(b_gd),
      W_p1, row(b_p1), W_p2, row(b_p2),
      W_po, row(b_po))
    return out


# speedup vs baseline: 1.0988x; 1.0988x over previous
import functools

import jax
import jax.numpy as jnp
from jax.experimental import pallas as pl
from jax.experimental.pallas import tpu as pltpu

N = 2048
A_OUT = 8
N_ATOMS = 51
V_MIN = -10.0
V_MAX = 10.0


N_CHUNKS = 8
CHUNK = N // N_CHUNKS


def _fused_kernel(feat_ref, adj_ref, mask_ref,
                  w_e1_ref, b_e1_ref, w_e2_ref, b_e2_ref,
                  w_g_ref, b_g_ref, w_gd_ref, b_gd_ref,
                  w_p1_ref, b_p1_ref, w_p2_ref, b_p2_ref,
                  w_po_ref, b_po_ref, out_ref, adj_vmem, sems):
    f32 = jnp.float32

    copies = []
    for k in range(N_CHUNKS):
        cp = pltpu.make_async_copy(
            adj_ref.at[pl.ds(k * CHUNK, CHUNK), :],
            adj_vmem.at[pl.ds(k * CHUNK, CHUNK), :],
            sems.at[k])
        cp.start()
        copies.append(cp)

    x = jnp.maximum(
        jnp.dot(feat_ref[...], w_e1_ref[...], preferred_element_type=f32)
        + b_e1_ref[...], 0.0)
    x = jnp.maximum(
        jnp.dot(x, w_e2_ref[...], preferred_element_type=f32)
        + b_e2_ref[...], 0.0)

    xw = jnp.dot(x, w_g_ref[...], preferred_element_type=f32)

    deg = jnp.zeros((1, N), f32) + 1.0
    for k in range(N_CHUNKS):
        copies[k].wait()
        deg = deg + jnp.sum(adj_vmem[pl.ds(k * CHUNK, CHUNK), :], axis=0,
                            keepdims=True)
    dinv = jax.lax.rsqrt(deg)
    z = xw * dinv.reshape(N, 1)

    agg = jax.lax.dot_general(
        adj_vmem[...], z, dimension_numbers=(((0,), (0,)), ((), ())),
        preferred_element_type=f32)
    gcn = dinv.reshape(N, 1) * (agg + z) + b_g_ref[...]
    xg = jnp.maximum(gcn, 0.0)
    xg = jnp.maximum(
        jnp.dot(xg, w_gd_ref[...], preferred_element_type=f32)
        + b_gd_ref[...], 0.0)

    p = jnp.maximum(
        jnp.dot(xg, w_p1_ref[0:32, :], preferred_element_type=f32)
        + jnp.dot(x, w_p1_ref[32:64, :], preferred_element_type=f32)
        + b_p1_ref[...], 0.0)
    p = jnp.maximum(
        jnp.dot(p, w_p2_ref[...], preferred_element_type=f32)
        + b_p2_ref[...], 0.0)
    p = (jnp.dot(p, w_po_ref[...], preferred_element_type=f32)
         + b_po_ref[...])
    mask_col = jnp.transpose(mask_ref[...])
    p = p * mask_col

    K = A_OUT * N_ATOMS
    step = (V_MAX - V_MIN) / (N_ATOMS - 1)
    k_i = jax.lax.broadcasted_iota(jnp.int32, (K, A_OUT), 0)
    a_i = jax.lax.broadcasted_iota(jnp.int32, (K, A_OUT), 1)
    G = (k_i // N_ATOMS == a_i).astype(f32)
    a_t = jax.lax.broadcasted_iota(jnp.int32, (A_OUT, K), 0)
    k_t = jax.lax.broadcasted_iota(jnp.int32, (A_OUT, K), 1)
    Gt = (k_t // N_ATOMS == a_t).astype(f32)
    sup_row = V_MIN + (jax.lax.broadcasted_iota(
        jnp.int32, (1, K), 1) % N_ATOMS).astype(f32) * step

    def dot2(a, b):
        a_hi = a.astype(jnp.bfloat16).astype(f32)
        return (jnp.dot(a_hi, b, preferred_element_type=f32)
                + jnp.dot(a - a_hi, b, preferred_element_type=f32))

    m = jnp.max(p, axis=1, keepdims=True)
    e = jnp.exp(p - m)
    denom = dot2(e, G)
    rden = 1.0 / denom
    d = e * dot2(rden, Gt)
    d = jnp.maximum(d, 0.001)
    out_ref[...] = dot2(d * sup_row, G)


@jax.jit
def kernel(features, adjacency, mask, W_e1, b_e1, W_e2, b_e2, W_g, b_g,
           W_gd, b_gd, W_p1, b_p1, W_p2, b_p2, W_po, b_po):
    mask2 = mask.reshape(1, N)
    row = lambda b: b.reshape(1, -1)
    n_in = 17
    in_specs = [pl.BlockSpec(memory_space=pltpu.MemorySpace.HBM) if i == 1
                else pl.BlockSpec(memory_space=pltpu.MemorySpace.VMEM)
                for i in range(n_in)]
    out = pl.pallas_call(
        _fused_kernel,
        out_shape=jax.ShapeDtypeStruct((N, A_OUT), jnp.float32),
        in_specs=in_specs,
        out_specs=pl.BlockSpec(memory_space=pltpu.MemorySpace.VMEM),
        scratch_shapes=[
            pltpu.VMEM((N, N), jnp.float32),
            pltpu.SemaphoreType.DMA((N_CHUNKS,)),
        ],
        compiler_params=pltpu.CompilerParams(
            vmem_limit_bytes=100 * 1024 * 1024),
    )(features, adjacency, mask2,
      W_e1, row(b_e1), W_e2, row(b_e2),
      W_g, row(b_g), W_gd, row(b_gd),
      W_p1, row(b_p1), W_p2, row(b_p2),
      W_po, row(b_po))
    return out
